# baseline (speedup 1.0000x reference)
import functools

import jax
import jax.numpy as jnp
from jax import lax
from jax.experimental import pallas as pl
from jax.experimental.pallas import tpu as pltpu
from jax.experimental.pallas import tpu_sc as plsc

E = 160000
T = 320000
D = 128
NR = 6
NSR = 42
NB = 8

SC_NC = 2
SC_NS = 16
SC_NW = SC_NC * SC_NS

f32 = jnp.float32


BE = 2000


def _pre_body(x_ref, rbf_ref, wrbf_ref, wkj_ref, bkj_ref, wji_ref, bji_ref,
              xkj_ref, xji_ref):
    x = x_ref[...]
    rbf_p = jnp.dot(rbf_ref[...], wrbf_ref[...], preferred_element_type=f32)
    xkj = jax.nn.silu(jnp.dot(x, wkj_ref[...], preferred_element_type=f32)
                      + bkj_ref[...])
    xji = jax.nn.silu(jnp.dot(x, wji_ref[...], preferred_element_type=f32)
                      + bji_ref[...])
    xkj_ref[...] = xkj * rbf_p
    xji_ref[...] = xji


def _pre_call(x, rbf, W_rbf, W_kj, b_kj, W_ji, b_ji):
    grid = (E // BE,)
    full = lambda shape: pl.BlockSpec(shape, lambda i: (0, 0))
    return pl.pallas_call(
        _pre_body,
        grid=grid,
        in_specs=[
            pl.BlockSpec((BE, D), lambda i: (i, 0)),
            pl.BlockSpec((BE, NR), lambda i: (i, 0)),
            full((NR, D)),
            full((D, D)),
            full((1, D)),
            full((D, D)),
            full((1, D)),
        ],
        out_specs=[
            pl.BlockSpec((BE, D), lambda i: (i, 0)),
            pl.BlockSpec((BE, D), lambda i: (i, 0)),
        ],
        out_shape=[
            jax.ShapeDtypeStruct((E, D), f32),
            jax.ShapeDtypeStruct((E, D), f32),
        ],
    )(x, rbf, W_rbf, W_kj, b_kj, W_ji, b_ji)


TH = T // 2
GW = 200
G_NWIN = TH // GW
G_WPW = G_NWIN // SC_NW


def _gather_body(table_hbm, idx_hbm, out_hbm, idx_v, rows_v, sem):
    c = lax.axis_index("c")
    s = lax.axis_index("s")
    wid = s * SC_NC + c
    for k in range(G_WPW):
        row = wid * G_WPW + k
        pltpu.sync_copy(idx_hbm.at[row], idx_v)
        pltpu.async_copy(table_hbm.at[idx_v], rows_v, sem).wait()
        pltpu.sync_copy(rows_v, out_hbm.at[pl.ds(row * GW, GW), :])


def _gather_call(table, idx2d):
    mesh = plsc.VectorSubcoreMesh(core_axis_name="c", subcore_axis_name="s",
                                  num_cores=SC_NC, num_subcores=SC_NS)
    k = functools.partial(
        pl.kernel,
        out_type=jax.ShapeDtypeStruct((TH, D), f32),
        mesh=mesh,
        scratch_types=[
            pltpu.VMEM((GW,), jnp.int32),
            pltpu.VMEM((GW, D), f32),
            pltpu.SemaphoreType.DMA,
        ],
    )(_gather_body)
    return k(table, idx2d)


BT = 4000


def _bil_body(g_ref, sbf_ref, wsbf_ref, wbt_ref, out_ref):
    sbf_p = jnp.dot(sbf_ref[...], wsbf_ref[...],
                    preferred_element_type=f32).astype(jnp.bfloat16)
    g = g_ref[...].astype(jnp.bfloat16)
    h = jnp.concatenate([sbf_p[:, i:i + 1] * g for i in range(NB)], axis=1)
    out_ref[...] = jnp.dot(h, wbt_ref[...],
                           preferred_element_type=f32).astype(jnp.bfloat16)


def _bil_call(g, sbf, W_sbf, W_bil_t):
    grid = (TH // BT,)
    return pl.pallas_call(
        _bil_body,
        grid=grid,
        in_specs=[
            pl.BlockSpec((BT, D), lambda i: (i, 0)),
            pl.BlockSpec((BT, NSR), lambda i: (i, 0)),
            pl.BlockSpec((NSR, NB), lambda i: (0, 0)),
            pl.BlockSpec((NB * D, D), lambda i: (0, 0)),
        ],
        out_specs=pl.BlockSpec((BT, D), lambda i: (i, 0)),
        out_shape=jax.ShapeDtypeStruct((TH, D), jnp.bfloat16),
    )(g, sbf, W_sbf, W_bil_t)


CCH = 16
NCHUNK = D // CCH
CH_PER_CORE = NCHUNK // SC_NC
SW = 2000
S_NWIN = TH // SW
S_WPS = S_NWIN // SC_NS
E_PS = E // SC_NS
Z_PER = E_PS // SW


ZW = 500
NZ = E_PS // ZW


def _scatter_body(bil_hbm, ji_hbm, zeros_hbm, out_hbm,
                  idx_v0, idx_v1, upd_v0, upd_v1, zbuf_v, acc_sh,
                  fsem, isem, zsem, wsem):
    c = lax.axis_index("c")
    s = lax.axis_index("s")
    pltpu.sync_copy(zeros_hbm, zbuf_v)
    upd = (upd_v0, upd_v1)
    idx = (idx_v0, idx_v1)

    def fetch(ch, w):
        col0 = (c * CH_PER_CORE + ch) * CCH
        row = s * S_WPS + w
        return (
            pltpu.async_copy(
                bil_hbm.at[pl.ds(row * SW, SW), pl.ds(col0, CCH)],
                upd[w % 2], fsem),
            pltpu.async_copy(ji_hbm.at[row], idx[w % 2], isem),
        )

    for ch_local in range(CH_PER_CORE):
        col0 = (c * CH_PER_CORE + ch_local) * CCH
        zc = [pltpu.async_copy(
                  zbuf_v, acc_sh.at[pl.ds(s * E_PS + z * ZW, ZW), :], zsem)
              for z in range(NZ)]
        for d in zc:
            d.wait()
        plsc.subcore_barrier()
        pend = fetch(ch_local, 0)
        for w in range(S_WPS):
            nxt = fetch(ch_local, w + 1) if w + 1 < S_WPS else None
            for d in pend:
                d.wait()
            pltpu.sync_copy(upd[w % 2], acc_sh.at[idx[w % 2]], add=True)
            pend = nxt
        plsc.subcore_barrier()
        wc = [pltpu.async_copy(
                  acc_sh.at[pl.ds(s * E_PS + z * SW, SW), :],
                  out_hbm.at[pl.ds(s * E_PS + z * SW, SW), pl.ds(col0, CCH)],
                  wsem)
              for z in range(Z_PER)]
        for d in wc:
            d.wait()
        plsc.subcore_barrier()


def _scatter_call(bil, ji2d, zeros_w):
    mesh = plsc.VectorSubcoreMesh(core_axis_name="c", subcore_axis_name="s",
                                  num_cores=SC_NC, num_subcores=SC_NS)
    k = functools.partial(
        pl.kernel,
        out_type=jax.ShapeDtypeStruct((E, D), jnp.bfloat16),
        mesh=mesh,
        scratch_types=[
            pltpu.VMEM((SW,), jnp.int32),
            pltpu.VMEM((SW,), jnp.int32),
            pltpu.VMEM((SW, CCH), jnp.bfloat16),
            pltpu.VMEM((SW, CCH), jnp.bfloat16),
            pltpu.VMEM((ZW, CCH), jnp.bfloat16),
            pltpu.VMEM_SHARED((E, CCH), jnp.bfloat16),
            pltpu.SemaphoreType.DMA,
            pltpu.SemaphoreType.DMA,
            pltpu.SemaphoreType.DMA,
            pltpu.SemaphoreType.DMA,
        ],
        compiler_params=pltpu.CompilerParams(use_tc_tiling_on_sc=False),
    )(_scatter_body)
    return k(bil, ji2d, zeros_w)


def _stack_body(xji_ref, agga_ref, aggb_ref, x_ref,
                w11, b11, w12, b12, wm, bm, w21, b21, w22, b22,
                w31, b31, w32, b32, out_ref):
    act = jax.nn.silu
    dot = lambda a, w: jnp.dot(a, w[...], preferred_element_type=f32)
    h = xji_ref[...] + (agga_ref[...].astype(f32) + aggb_ref[...].astype(f32))
    h = h + act(dot(act(dot(h, w11) + b11[...]), w12) + b12[...])
    h = act(dot(h, wm) + bm[...])
    h = h + x_ref[...]
    h = h + act(dot(act(dot(h, w21) + b21[...]), w22) + b22[...])
    h = h + act(dot(act(dot(h, w31) + b31[...]), w32) + b32[...])
    out_ref[...] = h


def _stack_call(x_ji, agg_a, agg_b, x, weights):
    grid = (E // BE,)
    blk = pl.BlockSpec((BE, D), lambda i: (i, 0))
    wspecs = []
    for w in weights:
        if w.shape == (D, D):
            wspecs.append(pl.BlockSpec((D, D), lambda i: (0, 0)))
        else:
            wspecs.append(pl.BlockSpec((1, D), lambda i: (0, 0)))
    return pl.pallas_call(
        _stack_body,
        grid=grid,
        in_specs=[blk, blk, blk, blk] + wspecs,
        out_specs=blk,
        out_shape=jax.ShapeDtypeStruct((E, D), f32),
    )(x_ji, agg_a, agg_b, x, *weights)


def kernel(x, rbf, sbf, edge_idx_kj, edge_idx_ji, W_rbf, W_sbf, W_kj, b_kj,
           W_ji, b_ji, W_bil, rb1_W1, rb1_b1, rb1_W2, rb1_b2, W_mid, b_mid,
           rb2_W1, rb2_b1, rb2_W2, rb2_b2, rb3_W1, rb3_b1, rb3_W2, rb3_b2):
    kj2d = edge_idx_kj.astype(jnp.int32).reshape(2, G_NWIN, GW)
    ji2d = edge_idx_ji.astype(jnp.int32).reshape(2, S_NWIN, SW)
    W_bil_t = jnp.transpose(W_bil, (1, 2, 0)).reshape(NB * D, D)
    W_bil_t = W_bil_t.astype(jnp.bfloat16)
    zeros_w = jnp.zeros((ZW, CCH), jnp.bfloat16)

    b2d = lambda b: b.reshape(1, D)

    x_kj, x_ji = _pre_call(x, rbf, W_rbf, W_kj, b2d(b_kj), W_ji, b2d(b_ji))
    sbf_a, sbf_b = sbf[:TH], sbf[TH:]
    g_a = _gather_call(x_kj, kj2d[0])
    g_b = _gather_call(x_kj, kj2d[1])
    bil_a = _bil_call(g_a, sbf_a, W_sbf, W_bil_t)
    agg_a = _scatter_call(bil_a, ji2d[0], zeros_w)
    bil_b = _bil_call(g_b, sbf_b, W_sbf, W_bil_t)
    agg_b = _scatter_call(bil_b, ji2d[1], zeros_w)
    weights = (rb1_W1, b2d(rb1_b1), rb1_W2, b2d(rb1_b2), W_mid, b2d(b_mid),
               rb2_W1, b2d(rb2_b1), rb2_W2, b2d(rb2_b2),
               rb3_W1, b2d(rb3_b1), rb3_W2, b2d(rb3_b2))
    return _stack_call(x_ji, agg_a, agg_b, x, weights)


# speedup vs baseline: 1.1876x; 1.1876x over previous
import functools

import jax
import jax.numpy as jnp
from jax import lax
from jax.experimental import pallas as pl
from jax.experimental.pallas import tpu as pltpu
from jax.experimental.pallas import tpu_sc as plsc

E = 160000
T = 320000
D = 128
NR = 6
NSR = 42
NB = 8

SC_NC = 2
SC_NS = 16
SC_NW = SC_NC * SC_NS

f32 = jnp.float32


BE = 2000


def _pre_body(x_ref, rbf_ref, wrbf_ref, wkj_ref, bkj_ref, wji_ref, bji_ref,
              xkj_ref, xji_ref):
    x = x_ref[...]
    rbf_p = jnp.dot(rbf_ref[...], wrbf_ref[...], preferred_element_type=f32)
    xkj = jax.nn.silu(jnp.dot(x, wkj_ref[...], preferred_element_type=f32)
                      + bkj_ref[...])
    xji = jax.nn.silu(jnp.dot(x, wji_ref[...], preferred_element_type=f32)
                      + bji_ref[...])
    xkj_ref[...] = xkj * rbf_p
    xji_ref[...] = xji


def _pre_call(x, rbf, W_rbf, W_kj, b_kj, W_ji, b_ji):
    grid = (E // BE,)
    full = lambda shape: pl.BlockSpec(shape, lambda i: (0, 0))
    return pl.pallas_call(
        _pre_body,
        grid=grid,
        in_specs=[
            pl.BlockSpec((BE, D), lambda i: (i, 0)),
            pl.BlockSpec((BE, NR), lambda i: (i, 0)),
            full((NR, D)),
            full((D, D)),
            full((1, D)),
            full((D, D)),
            full((1, D)),
        ],
        out_specs=[
            pl.BlockSpec((BE, D), lambda i: (i, 0)),
            pl.BlockSpec((BE, D), lambda i: (i, 0)),
        ],
        out_shape=[
            jax.ShapeDtypeStruct((E, D), f32),
            jax.ShapeDtypeStruct((E, D), f32),
        ],
    )(x, rbf, W_rbf, W_kj, b_kj, W_ji, b_ji)


GW = 400
G_NWIN = T // GW
G_WPW = G_NWIN // SC_NW


def _gather_body(table_hbm, idx_hbm, out_hbm, idx_v, rows_v, sem):
    c = lax.axis_index("c")
    s = lax.axis_index("s")
    wid = s * SC_NC + c
    for k in range(G_WPW):
        row = wid * G_WPW + k
        pltpu.sync_copy(idx_hbm.at[row], idx_v)
        pltpu.async_copy(table_hbm.at[idx_v], rows_v, sem).wait()
        pltpu.sync_copy(rows_v, out_hbm.at[pl.ds(row * GW, GW), :])


def _gather_call(table, idx2d):
    mesh = plsc.VectorSubcoreMesh(core_axis_name="c", subcore_axis_name="s",
                                  num_cores=SC_NC, num_subcores=SC_NS)
    k = functools.partial(
        pl.kernel,
        out_type=jax.ShapeDtypeStruct((T, D), f32),
        mesh=mesh,
        scratch_types=[
            pltpu.VMEM((GW,), jnp.int32),
            pltpu.VMEM((GW, D), f32),
            pltpu.SemaphoreType.DMA,
        ],
    )(_gather_body)
    return k(table, idx2d)


BT = 4000


def _bil_body(g_ref, sbf_ref, wsbf_ref, wbt_ref, out_ref):
    sbf_p = jnp.dot(sbf_ref[...], wsbf_ref[...],
                    preferred_element_type=f32).astype(jnp.bfloat16)
    g = g_ref[...].astype(jnp.bfloat16)
    h = jnp.concatenate([sbf_p[:, i:i + 1] * g for i in range(NB)], axis=1)
    out_ref[...] = jnp.dot(h, wbt_ref[...],
                           preferred_element_type=f32).astype(jnp.bfloat16)


def _bil_call(g, sbf, W_sbf, W_bil_t):
    grid = (T // BT,)
    return pl.pallas_call(
        _bil_body,
        grid=grid,
        in_specs=[
            pl.BlockSpec((BT, D), lambda i: (i, 0)),
            pl.BlockSpec((BT, NSR), lambda i: (i, 0)),
            pl.BlockSpec((NSR, NB), lambda i: (0, 0)),
            pl.BlockSpec((NB * D, D), lambda i: (0, 0)),
        ],
        out_specs=pl.BlockSpec((BT, D), lambda i: (i, 0)),
        out_shape=jax.ShapeDtypeStruct((T, D), jnp.bfloat16),
    )(g, sbf, W_sbf, W_bil_t)


CCH = 16
NCHUNK = D // CCH
CH_PER_CORE = NCHUNK // SC_NC
SW = 2000
S_NWIN = T // SW
S_WPS = S_NWIN // SC_NS
E_PS = E // SC_NS
Z_PER = E_PS // SW


ZW = 500
NZ = E_PS // ZW


def _scatter_body(bil_hbm, ji_hbm, zeros_hbm, out_hbm,
                  idx_v0, idx_v1, upd_v0, upd_v1, zbuf_v, acc_sh,
                  fsem, isem, zsem, wsem):
    c = lax.axis_index("c")
    s = lax.axis_index("s")
    pltpu.sync_copy(zeros_hbm, zbuf_v)
    upd = (upd_v0, upd_v1)
    idx = (idx_v0, idx_v1)

    def fetch(ch, w):
        col0 = (c * CH_PER_CORE + ch) * CCH
        row = s * S_WPS + w
        return (
            pltpu.async_copy(
                bil_hbm.at[pl.ds(row * SW, SW), pl.ds(col0, CCH)],
                upd[w % 2], fsem),
            pltpu.async_copy(ji_hbm.at[row], idx[w % 2], isem),
        )

    for ch_local in range(CH_PER_CORE):
        col0 = (c * CH_PER_CORE + ch_local) * CCH
        zc = [pltpu.async_copy(
                  zbuf_v, acc_sh.at[pl.ds(s * E_PS + z * ZW, ZW), :], zsem)
              for z in range(NZ)]
        for d in zc:
            d.wait()
        plsc.subcore_barrier()
        pend = fetch(ch_local, 0)
        for w in range(S_WPS):
            nxt = fetch(ch_local, w + 1) if w + 1 < S_WPS else None
            for d in pend:
                d.wait()
            pltpu.sync_copy(upd[w % 2], acc_sh.at[idx[w % 2]], add=True)
            pend = nxt
        plsc.subcore_barrier()
        wc = [pltpu.async_copy(
                  acc_sh.at[pl.ds(s * E_PS + z * SW, SW), :],
                  out_hbm.at[pl.ds(s * E_PS + z * SW, SW), pl.ds(col0, CCH)],
                  wsem)
              for z in range(Z_PER)]
        for d in wc:
            d.wait()
        plsc.subcore_barrier()


def _scatter_call(bil, ji2d, zeros_w):
    mesh = plsc.VectorSubcoreMesh(core_axis_name="c", subcore_axis_name="s",
                                  num_cores=SC_NC, num_subcores=SC_NS)
    k = functools.partial(
        pl.kernel,
        out_type=jax.ShapeDtypeStruct((E, D), jnp.bfloat16),
        mesh=mesh,
        scratch_types=[
            pltpu.VMEM((SW,), jnp.int32),
            pltpu.VMEM((SW,), jnp.int32),
            pltpu.VMEM((SW, CCH), jnp.bfloat16),
            pltpu.VMEM((SW, CCH), jnp.bfloat16),
            pltpu.VMEM((ZW, CCH), jnp.bfloat16),
            pltpu.VMEM_SHARED((E, CCH), jnp.bfloat16),
            pltpu.SemaphoreType.DMA,
            pltpu.SemaphoreType.DMA,
            pltpu.SemaphoreType.DMA,
            pltpu.SemaphoreType.DMA,
        ],
        compiler_params=pltpu.CompilerParams(use_tc_tiling_on_sc=False),
    )(_scatter_body)
    return k(bil, ji2d, zeros_w)


def _stack_body(xji_ref, agg_ref, x_ref,
                w11, b11, w12, b12, wm, bm, w21, b21, w22, b22,
                w31, b31, w32, b32, out_ref):
    act = jax.nn.silu
    dot = lambda a, w: jnp.dot(a, w[...], preferred_element_type=f32)
    h = xji_ref[...] + agg_ref[...].astype(f32)
    h = h + act(dot(act(dot(h, w11) + b11[...]), w12) + b12[...])
    h = act(dot(h, wm) + bm[...])
    h = h + x_ref[...]
    h = h + act(dot(act(dot(h, w21) + b21[...]), w22) + b22[...])
    h = h + act(dot(act(dot(h, w31) + b31[...]), w32) + b32[...])
    out_ref[...] = h


def _stack_call(x_ji, agg, x, weights):
    grid = (E // BE,)
    blk = pl.BlockSpec((BE, D), lambda i: (i, 0))
    wspecs = []
    for w in weights:
        if w.shape == (D, D):
            wspecs.append(pl.BlockSpec((D, D), lambda i: (0, 0)))
        else:
            wspecs.append(pl.BlockSpec((1, D), lambda i: (0, 0)))
    return pl.pallas_call(
        _stack_body,
        grid=grid,
        in_specs=[blk, blk, blk] + wspecs,
        out_specs=blk,
        out_shape=jax.ShapeDtypeStruct((E, D), f32),
    )(x_ji, agg, x, *weights)


def kernel(x, rbf, sbf, edge_idx_kj, edge_idx_ji, W_rbf, W_sbf, W_kj, b_kj,
           W_ji, b_ji, W_bil, rb1_W1, rb1_b1, rb1_W2, rb1_b2, W_mid, b_mid,
           rb2_W1, rb2_b1, rb2_W2, rb2_b2, rb3_W1, rb3_b1, rb3_W2, rb3_b2):
    kj2d = edge_idx_kj.astype(jnp.int32).reshape(G_NWIN, GW)
    ji2d = edge_idx_ji.astype(jnp.int32).reshape(S_NWIN, SW)
    W_bil_t = jnp.transpose(W_bil, (1, 2, 0)).reshape(NB * D, D)
    W_bil_t = W_bil_t.astype(jnp.bfloat16)
    zeros_w = jnp.zeros((ZW, CCH), jnp.bfloat16)

    b2d = lambda b: b.reshape(1, D)

    x_kj, x_ji = _pre_call(x, rbf, W_rbf, W_kj, b2d(b_kj), W_ji, b2d(b_ji))
    g = _gather_call(x_kj, kj2d)
    bil = _bil_call(g, sbf, W_sbf, W_bil_t)
    agg = _scatter_call(bil, ji2d, zeros_w)
    weights = (rb1_W1, b2d(rb1_b1), rb1_W2, b2d(rb1_b2), W_mid, b2d(b_mid),
               rb2_W1, b2d(rb2_b1), rb2_W2, b2d(rb2_b2),
               rb3_W1, b2d(rb3_b1), rb3_W2, b2d(rb3_b2))
    return _stack_call(x_ji, agg, x, weights)


# speedup vs baseline: 1.1964x; 1.0074x over previous
import functools

import jax
import jax.numpy as jnp
from jax import lax
from jax.experimental import pallas as pl
from jax.experimental.pallas import tpu as pltpu
from jax.experimental.pallas import tpu_sc as plsc

E = 160000
T = 320000
D = 128
NR = 6
NSR = 42
NB = 8

SC_NC = 2
SC_NS = 16
SC_NW = SC_NC * SC_NS

f32 = jnp.float32


BE = 2000


def _prekj_body(x_ref, rbf_ref, wrbf_ref, wkj_ref, bkj_ref, xkj_ref):
    x = x_ref[...]
    rbf_p = jnp.dot(rbf_ref[...], wrbf_ref[...], preferred_element_type=f32)
    xkj = jax.nn.silu(jnp.dot(x, wkj_ref[...], preferred_element_type=f32)
                      + bkj_ref[...])
    xkj_ref[...] = xkj * rbf_p


def _prekj_call(x, rbf, W_rbf, W_kj, b_kj):
    grid = (E // BE,)
    full = lambda shape: pl.BlockSpec(shape, lambda i: (0, 0))
    return pl.pallas_call(
        _prekj_body,
        grid=grid,
        in_specs=[
            pl.BlockSpec((BE, D), lambda i: (i, 0)),
            pl.BlockSpec((BE, NR), lambda i: (i, 0)),
            full((NR, D)),
            full((D, D)),
            full((1, D)),
        ],
        out_specs=pl.BlockSpec((BE, D), lambda i: (i, 0)),
        out_shape=jax.ShapeDtypeStruct((E, D), f32),
    )(x, rbf, W_rbf, W_kj, b_kj)


def _preji_body(x_ref, wji_ref, bji_ref, xji_ref):
    xji_ref[...] = jax.nn.silu(
        jnp.dot(x_ref[...], wji_ref[...], preferred_element_type=f32)
        + bji_ref[...])


def _preji_call(x, W_ji, b_ji):
    grid = (E // BE,)
    full = lambda shape: pl.BlockSpec(shape, lambda i: (0, 0))
    return pl.pallas_call(
        _preji_body,
        grid=grid,
        in_specs=[
            pl.BlockSpec((BE, D), lambda i: (i, 0)),
            full((D, D)),
            full((1, D)),
        ],
        out_specs=pl.BlockSpec((BE, D), lambda i: (i, 0)),
        out_shape=jax.ShapeDtypeStruct((E, D), f32),
    )(x, W_ji, b_ji)


GW = 400
G_NWIN = T // GW
G_WPW = G_NWIN // SC_NW


def _gather_body(table_hbm, idx_hbm, out_hbm, idx_v, rows_v, sem):
    c = lax.axis_index("c")
    s = lax.axis_index("s")
    wid = s * SC_NC + c
    for k in range(G_WPW):
        row = wid * G_WPW + k
        pltpu.sync_copy(idx_hbm.at[row], idx_v)
        pltpu.async_copy(table_hbm.at[idx_v], rows_v, sem).wait()
        pltpu.sync_copy(rows_v, out_hbm.at[pl.ds(row * GW, GW), :])


def _gather_call(table, idx2d):
    mesh = plsc.VectorSubcoreMesh(core_axis_name="c", subcore_axis_name="s",
                                  num_cores=SC_NC, num_subcores=SC_NS)
    k = functools.partial(
        pl.kernel,
        out_type=jax.ShapeDtypeStruct((T, D), f32),
        mesh=mesh,
        scratch_types=[
            pltpu.VMEM((GW,), jnp.int32),
            pltpu.VMEM((GW, D), f32),
            pltpu.SemaphoreType.DMA,
        ],
    )(_gather_body)
    return k(table, idx2d)


BT = 4000


def _bil_body(g_ref, sbf_ref, wsbf_ref, wbt_ref, out_ref):
    sbf_p = jnp.dot(sbf_ref[...], wsbf_ref[...],
                    preferred_element_type=f32).astype(jnp.bfloat16)
    g = g_ref[...].astype(jnp.bfloat16)
    h = jnp.concatenate([sbf_p[:, i:i + 1] * g for i in range(NB)], axis=1)
    out_ref[...] = jnp.dot(h, wbt_ref[...],
                           preferred_element_type=f32).astype(jnp.bfloat16)


def _bil_call(g, sbf, W_sbf, W_bil_t):
    grid = (T // BT,)
    return pl.pallas_call(
        _bil_body,
        grid=grid,
        in_specs=[
            pl.BlockSpec((BT, D), lambda i: (i, 0)),
            pl.BlockSpec((BT, NSR), lambda i: (i, 0)),
            pl.BlockSpec((NSR, NB), lambda i: (0, 0)),
            pl.BlockSpec((NB * D, D), lambda i: (0, 0)),
        ],
        out_specs=pl.BlockSpec((BT, D), lambda i: (i, 0)),
        out_shape=jax.ShapeDtypeStruct((T, D), jnp.bfloat16),
    )(g, sbf, W_sbf, W_bil_t)


CCH = 16
NCHUNK = D // CCH
CH_PER_CORE = NCHUNK // SC_NC
SW = 2000
S_NWIN = T // SW
S_WPS = S_NWIN // SC_NS
E_PS = E // SC_NS
Z_PER = E_PS // SW


ZW = 500
NZ = E_PS // ZW


def _scatter_body(bil_hbm, ji_hbm, zeros_hbm, out_hbm,
                  idx_v0, idx_v1, upd_v0, upd_v1, zbuf_v, acc_sh,
                  fsem, isem, zsem, wsem, ssem):
    c = lax.axis_index("c")
    s = lax.axis_index("s")
    pltpu.sync_copy(zeros_hbm, zbuf_v)
    upd = (upd_v0, upd_v1)
    idx = (idx_v0, idx_v1)

    def fetch(ch, w):
        col0 = (c * CH_PER_CORE + ch) * CCH
        row = s * S_WPS + w
        return (
            pltpu.async_copy(
                bil_hbm.at[pl.ds(row * SW, SW), pl.ds(col0, CCH)],
                upd[w % 2], fsem),
            pltpu.async_copy(ji_hbm.at[row], idx[w % 2], isem),
        )

    for ch_local in range(CH_PER_CORE):
        col0 = (c * CH_PER_CORE + ch_local) * CCH
        zc = [pltpu.async_copy(
                  zbuf_v, acc_sh.at[pl.ds(s * E_PS + z * ZW, ZW), :], zsem)
              for z in range(NZ)]
        for d in zc:
            d.wait()
        plsc.subcore_barrier()
        pend_f = {0: fetch(ch_local, 0)}
        pend_s = {}
        for w in range(S_WPS):
            b = w % 2
            for d in pend_f.pop(w):
                d.wait()
            pend_s[b] = pltpu.async_copy(
                upd[b], acc_sh.at[idx[b]], ssem, add=True)
            if w + 1 < S_WPS:
                ob = (w + 1) % 2
                if ob in pend_s:
                    pend_s.pop(ob).wait()
                pend_f[w + 1] = fetch(ch_local, w + 1)
        for d in pend_s.values():
            d.wait()
        plsc.subcore_barrier()
        wc = [pltpu.async_copy(
                  acc_sh.at[pl.ds(s * E_PS + z * SW, SW), :],
                  out_hbm.at[pl.ds(s * E_PS + z * SW, SW), pl.ds(col0, CCH)],
                  wsem)
              for z in range(Z_PER)]
        for d in wc:
            d.wait()
        plsc.subcore_barrier()


def _scatter_call(bil, ji2d, zeros_w):
    mesh = plsc.VectorSubcoreMesh(core_axis_name="c", subcore_axis_name="s",
                                  num_cores=SC_NC, num_subcores=SC_NS)
    k = functools.partial(
        pl.kernel,
        out_type=jax.ShapeDtypeStruct((E, D), jnp.bfloat16),
        mesh=mesh,
        scratch_types=[
            pltpu.VMEM((SW,), jnp.int32),
            pltpu.VMEM((SW,), jnp.int32),
            pltpu.VMEM((SW, CCH), jnp.bfloat16),
            pltpu.VMEM((SW, CCH), jnp.bfloat16),
            pltpu.VMEM((ZW, CCH), jnp.bfloat16),
            pltpu.VMEM_SHARED((E, CCH), jnp.bfloat16),
            pltpu.SemaphoreType.DMA,
            pltpu.SemaphoreType.DMA,
            pltpu.SemaphoreType.DMA,
            pltpu.SemaphoreType.DMA,
            pltpu.SemaphoreType.DMA,
        ],
        compiler_params=pltpu.CompilerParams(use_tc_tiling_on_sc=False),
    )(_scatter_body)
    return k(bil, ji2d, zeros_w)


def _stack_body(xji_ref, agg_ref, x_ref,
                w11, b11, w12, b12, wm, bm, w21, b21, w22, b22,
                w31, b31, w32, b32, out_ref):
    act = jax.nn.silu
    dot = lambda a, w: jnp.dot(a, w[...], preferred_element_type=f32)
    h = xji_ref[...] + agg_ref[...].astype(f32)
    h = h + act(dot(act(dot(h, w11) + b11[...]), w12) + b12[...])
    h = act(dot(h, wm) + bm[...])
    h = h + x_ref[...]
    h = h + act(dot(act(dot(h, w21) + b21[...]), w22) + b22[...])
    h = h + act(dot(act(dot(h, w31) + b31[...]), w32) + b32[...])
    out_ref[...] = h


def _stack_call(x_ji, agg, x, weights):
    grid = (E // BE,)
    blk = pl.BlockSpec((BE, D), lambda i: (i, 0))
    wspecs = []
    for w in weights:
        if w.shape == (D, D):
            wspecs.append(pl.BlockSpec((D, D), lambda i: (0, 0)))
        else:
            wspecs.append(pl.BlockSpec((1, D), lambda i: (0, 0)))
    return pl.pallas_call(
        _stack_body,
        grid=grid,
        in_specs=[blk, blk, blk] + wspecs,
        out_specs=blk,
        out_shape=jax.ShapeDtypeStruct((E, D), f32),
    )(x_ji, agg, x, *weights)


def kernel(x, rbf, sbf, edge_idx_kj, edge_idx_ji, W_rbf, W_sbf, W_kj, b_kj,
           W_ji, b_ji, W_bil, rb1_W1, rb1_b1, rb1_W2, rb1_b2, W_mid, b_mid,
           rb2_W1, rb2_b1, rb2_W2, rb2_b2, rb3_W1, rb3_b1, rb3_W2, rb3_b2):
    kj2d = edge_idx_kj.astype(jnp.int32).reshape(G_NWIN, GW)
    ji2d = edge_idx_ji.astype(jnp.int32).reshape(S_NWIN, SW)
    W_bil_t = jnp.transpose(W_bil, (1, 2, 0)).reshape(NB * D, D)
    W_bil_t = W_bil_t.astype(jnp.bfloat16)
    zeros_w = jnp.zeros((ZW, CCH), jnp.bfloat16)

    b2d = lambda b: b.reshape(1, D)

    x_kj = _prekj_call(x, rbf, W_rbf, W_kj, b2d(b_kj))
    g = _gather_call(x_kj, kj2d)
    bil = _bil_call(g, sbf, W_sbf, W_bil_t)
    agg = _scatter_call(bil, ji2d, zeros_w)
    x_ji = _preji_call(x, W_ji, b2d(b_ji))
    weights = (rb1_W1, b2d(rb1_b1), rb1_W2, b2d(rb1_b2), W_mid, b2d(b_mid),
               rb2_W1, b2d(rb2_b1), rb2_W2, b2d(rb2_b2),
               rb3_W1, b2d(rb3_b1), rb3_W2, b2d(rb3_b2))
    return _stack_call(x_ji, agg, x, weights)
